# baseline (speedup 1.0000x reference)
import functools

import jax
import jax.numpy as jnp
from jax.experimental import pallas as pl
from jax.experimental.pallas import tpu as pltpu

_NUM_EMB = 1024
_DIM = 64
_COMMIT = 0.25


def _vq_body(flat_ref, embed_ref, x2_ref, e2_ref,
             q_ref, idx_ref, loss_ref, perp_ref, counts_ref,
             *, m_rows, n_total):
    step = pl.program_id(0)
    last = pl.num_programs(0) - 1

    f = flat_ref[...]
    e = embed_ref[...]
    m = jax.lax.dot_general(f, e, (((1,), (1,)), ((), ())),
                            preferred_element_type=jnp.float32)
    d = x2_ref[...] + e2_ref[...] - 2.0 * m

    dmin = jnp.min(d, axis=1, keepdims=True)
    lanes = jax.lax.broadcasted_iota(jnp.int32, (m_rows, _NUM_EMB), 1)
    idxv = jnp.min(jnp.where(d == dmin, lanes, _NUM_EMB), axis=1,
                   keepdims=True)
    idx_ref[...] = idxv

    onehot = (lanes == idxv).astype(jnp.float32)
    q = jax.lax.dot_general(onehot, e, (((1,), (0,)), ((), ())),
                            preferred_element_type=jnp.float32)
    q_ref[...] = q

    diff = q - f
    part_loss = jnp.sum(diff * diff).reshape(1, 1)
    part_counts = jnp.sum(onehot, axis=0, keepdims=True)

    @pl.when(step == 0)
    def _init():
        loss_ref[...] = jnp.zeros((1, 1), jnp.float32)
        counts_ref[...] = jnp.zeros_like(counts_ref)

    loss_ref[...] += part_loss
    counts_ref[...] += part_counts

    @pl.when(step == last)
    def _finish():
        p = counts_ref[...] / jnp.float32(n_total)
        perp_ref[...] = jnp.exp(-jnp.sum(p * jnp.log(p + 1e-10))).reshape(1, 1)
        loss_ref[...] = loss_ref[...] * jnp.float32(_COMMIT / (n_total * _DIM))


def _vq_tc(flat, embed, x2, e2, *, m_rows=512, interpret=False):
    n = flat.shape[0]
    grid = (n // m_rows,)
    out_shapes = (
        jax.ShapeDtypeStruct((n, _DIM), jnp.float32),
        jax.ShapeDtypeStruct((n, 1), jnp.int32),
        jax.ShapeDtypeStruct((1, 1), jnp.float32),
        jax.ShapeDtypeStruct((1, 1), jnp.float32),
    )
    return pl.pallas_call(
        functools.partial(_vq_body, m_rows=m_rows, n_total=n),
        grid=grid,
        in_specs=[
            pl.BlockSpec((m_rows, _DIM), lambda i: (i, 0)),
            pl.BlockSpec((_NUM_EMB, _DIM), lambda i: (0, 0)),
            pl.BlockSpec((m_rows, 1), lambda i: (i, 0)),
            pl.BlockSpec((1, _NUM_EMB), lambda i: (0, 0)),
        ],
        out_specs=[
            pl.BlockSpec((m_rows, _DIM), lambda i: (i, 0)),
            pl.BlockSpec((m_rows, 1), lambda i: (i, 0)),
            pl.BlockSpec((1, 1), lambda i: (0, 0)),
            pl.BlockSpec((1, 1), lambda i: (0, 0)),
        ],
        out_shape=out_shapes,
        scratch_shapes=[pltpu.VMEM((1, _NUM_EMB), jnp.float32)],
        interpret=interpret,
    )(flat, embed, x2, e2)


def kernel(inputs, embed):
    x = jnp.transpose(inputs, (0, 2, 3, 1))
    B, H, W, C = x.shape
    flat = x.reshape(-1, _DIM).astype(jnp.float32)
    embed_f = embed.astype(jnp.float32)
    x2 = jnp.sum(flat ** 2, axis=1, keepdims=True)
    e2 = jnp.sum(embed_f ** 2, axis=1)[None, :]

    q, idx, loss, perp = _vq_tc(flat, embed_f, x2, e2)

    quantized_out = jnp.transpose(q.reshape(B, H, W, C), (0, 3, 1, 2))
    quantized_out = quantized_out.astype(inputs.dtype)
    encoding_indices = idx.reshape(B, H, W)
    return (quantized_out, loss[0, 0], perp[0, 0], encoding_indices)
